# Initial kernel scaffold; baseline (speedup 1.0000x reference)
import jax
import jax.numpy as jnp
from jax.experimental import pallas as pl


def kernel(inp, Wh, bh, W1a, b1a, W1b, b1b, W2a, b2a, W2b, b2b):
    raise NotImplementedError("write your pallas kernel here")


# speedup vs baseline: 1.2446x; 1.2446x over previous
import functools

import jax
import jax.numpy as jnp
from jax import lax
from jax.experimental import pallas as pl
from jax.experimental.pallas import tpu as pltpu

S = 2048
D = 768
H = 20002
V = 40000
P1 = 192
P2 = 48
HEAD_END = 20000
T1_END = 60000
OUT_W = 100000

CT = 2560
R = 512
I = S // R
JH = 8
JT = 16
JB = 40
J_T1 = 7
J_T2 = 23
NEG = -1e30


def _head_stats_kernel(x_ref, w_ref, b_ref, wg_ref, bg_ref,
                       c_ref, g_ref, m_scr, s_scr):
    j = pl.program_id(0)
    i = pl.program_id(1)
    rows = pl.ds(i * R, R)

    @pl.when(j == 0)
    def _():
        m_scr[rows, :] = jnp.full((R, 1), NEG, jnp.float32)
        s_scr[rows, :] = jnp.zeros((R, 1), jnp.float32)

    l = jnp.dot(x_ref[...], w_ref[...],
                preferred_element_type=jnp.float32) + b_ref[...]
    col = j * CT + lax.broadcasted_iota(jnp.int32, (R, CT), 1)
    lm = jnp.where(col < H, l, NEG)
    m_old = m_scr[rows, :]
    m_new = jnp.maximum(m_old, jnp.max(lm, axis=1, keepdims=True))
    s_new = (s_scr[rows, :] * jnp.exp(m_old - m_new)
             + jnp.sum(jnp.exp(lm - m_new), axis=1, keepdims=True))
    m_scr[rows, :] = m_new
    s_scr[rows, :] = s_new

    @pl.when(j == JH - 1)
    def _():
        c = m_new + jnp.log(s_new)
        c_ref[...] = c
        lg = jnp.dot(x_ref[...], wg_ref[...],
                     preferred_element_type=jnp.float32) + bg_ref[...]
        g_ref[...] = jnp.exp(lg - c)


def _tails_stats_kernel(x_ref, w1a_ref, b1a_ref, w1b_ref, b1b_ref,
                        w2a_ref, b2a_ref, w2b_ref, b2b_ref,
                        c2_ref, c3_ref, p1_ref, p2_ref,
                        m1_scr, s1_scr, m2_scr, s2_scr, p1_scr, p2_scr):
    j = pl.program_id(0)
    i = pl.program_id(1)
    rows = pl.ds(i * R, R)

    @pl.when(j == 0)
    def _():
        p1 = jnp.dot(x_ref[...], w1a_ref[...],
                     preferred_element_type=jnp.float32) + b1a_ref[...]
        p2 = jnp.dot(x_ref[...], w2a_ref[...],
                     preferred_element_type=jnp.float32) + b2a_ref[...]
        p1b = p1.astype(jnp.bfloat16)
        p2b = p2.astype(jnp.bfloat16)
        p1_scr[rows, :] = p1b
        p2_scr[rows, :] = p2b
        p1_ref[...] = p1b
        p2_ref[...] = p2b
        m1_scr[rows, :] = jnp.full((R, 1), NEG, jnp.float32)
        s1_scr[rows, :] = jnp.zeros((R, 1), jnp.float32)
        m2_scr[rows, :] = jnp.full((R, 1), NEG, jnp.float32)
        s2_scr[rows, :] = jnp.zeros((R, 1), jnp.float32)

    col = j * CT + lax.broadcasted_iota(jnp.int32, (R, CT), 1)
    valid = col < V

    l1 = jnp.dot(p1_scr[rows, :], w1b_ref[...],
                 preferred_element_type=jnp.float32) + b1b_ref[...]
    l1 = jnp.where(valid, l1, NEG)
    m1_old = m1_scr[rows, :]
    m1_new = jnp.maximum(m1_old, jnp.max(l1, axis=1, keepdims=True))
    s1_new = (s1_scr[rows, :] * jnp.exp(m1_old - m1_new)
              + jnp.sum(jnp.exp(l1 - m1_new), axis=1, keepdims=True))
    m1_scr[rows, :] = m1_new
    s1_scr[rows, :] = s1_new

    l2 = jnp.dot(p2_scr[rows, :], w2b_ref[...],
                 preferred_element_type=jnp.float32) + b2b_ref[...]
    l2 = jnp.where(valid, l2, NEG)
    m2_old = m2_scr[rows, :]
    m2_new = jnp.maximum(m2_old, jnp.max(l2, axis=1, keepdims=True))
    s2_new = (s2_scr[rows, :] * jnp.exp(m2_old - m2_new)
              + jnp.sum(jnp.exp(l2 - m2_new), axis=1, keepdims=True))
    m2_scr[rows, :] = m2_new
    s2_scr[rows, :] = s2_new

    @pl.when(j == JT - 1)
    def _():
        c2_ref[...] = m1_new + jnp.log(s1_new)
        c3_ref[...] = m2_new + jnp.log(s2_new)


def _out_kernel(x_ref, wh_ref, bh_ref, p1_ref, w1_ref, b1_ref,
                p2_ref, w2_ref, b2_ref, st_ref, o_ref):
    j = pl.program_id(0)
    c1 = st_ref[:, 0:1]
    c2 = st_ref[:, 1:2]
    c3 = st_ref[:, 2:3]
    g1 = st_ref[:, 3:4]
    g2 = st_ref[:, 4:5]

    def head_vals():
        l = jnp.dot(x_ref[...], wh_ref[...],
                    preferred_element_type=jnp.float32) + bh_ref[...]
        return jnp.exp(l - c1)

    def tail1_vals():
        l = jnp.dot(p1_ref[...], w1_ref[...],
                    preferred_element_type=jnp.float32) + b1_ref[...]
        return g1 * jnp.exp(l - c2)

    def tail2_vals():
        l = jnp.dot(p2_ref[...], w2_ref[...],
                    preferred_element_type=jnp.float32) + b2_ref[...]
        return g2 * jnp.exp(l - c3)

    @pl.when(j < J_T1)
    def _():
        o_ref[...] = head_vals()

    @pl.when(j == J_T1)
    def _():
        col = j * CT + lax.broadcasted_iota(jnp.int32, (R, CT), 1)
        o_ref[...] = jnp.where(col < HEAD_END, head_vals(), tail1_vals())

    @pl.when(jnp.logical_and(j > J_T1, j < J_T2))
    def _():
        o_ref[...] = tail1_vals()

    @pl.when(j == J_T2)
    def _():
        col = j * CT + lax.broadcasted_iota(jnp.int32, (R, CT), 1)
        o_ref[...] = jnp.where(col < T1_END, tail1_vals(), tail2_vals())

    @pl.when(j > J_T2)
    def _():
        o_ref[...] = tail2_vals()


@functools.partial(jax.jit, static_argnames=("interpret",))
def _run(inp, Wh, bh, W1a, b1a, W1b, b1b, W2a, b2a, W2b, b2b,
         interpret=False):
    x = inp.reshape(S, D).astype(jnp.bfloat16)
    whb = Wh.astype(jnp.bfloat16)
    w1ab = W1a.astype(jnp.bfloat16)
    w1bb = W1b.astype(jnp.bfloat16)
    w2ab = W2a.astype(jnp.bfloat16)
    w2bb = W2b.astype(jnp.bfloat16)
    bh2 = bh.reshape(1, H)
    b1a2 = b1a.reshape(1, P1)
    b2a2 = b2a.reshape(1, P2)
    b1b2 = b1b.reshape(1, V)
    b2b2 = b2b.reshape(1, V)
    whg = whb[:, HEAD_END:H]
    bhg = bh2[:, HEAD_END:H]

    f32 = jnp.float32
    c1, g = pl.pallas_call(
        _head_stats_kernel,
        grid=(JH, I),
        in_specs=[
            pl.BlockSpec((R, D), lambda j, i: (i, 0)),
            pl.BlockSpec((D, CT), lambda j, i: (0, j)),
            pl.BlockSpec((1, CT), lambda j, i: (0, j)),
            pl.BlockSpec((D, 2), lambda j, i: (0, 0)),
            pl.BlockSpec((1, 2), lambda j, i: (0, 0)),
        ],
        out_specs=[
            pl.BlockSpec((R, 1), lambda j, i: (i, 0)),
            pl.BlockSpec((R, 2), lambda j, i: (i, 0)),
        ],
        out_shape=[
            jax.ShapeDtypeStruct((S, 1), f32),
            jax.ShapeDtypeStruct((S, 2), f32),
        ],
        scratch_shapes=[
            pltpu.VMEM((S, 1), f32),
            pltpu.VMEM((S, 1), f32),
        ],
        interpret=interpret,
    )(x, whb, bh2, whg, bhg)

    c2, c3, p1, p2 = pl.pallas_call(
        _tails_stats_kernel,
        grid=(JT, I),
        in_specs=[
            pl.BlockSpec((R, D), lambda j, i: (i, 0)),
            pl.BlockSpec((D, P1), lambda j, i: (0, 0)),
            pl.BlockSpec((1, P1), lambda j, i: (0, 0)),
            pl.BlockSpec((P1, CT), lambda j, i: (0, j)),
            pl.BlockSpec((1, CT), lambda j, i: (0, j)),
            pl.BlockSpec((D, P2), lambda j, i: (0, 0)),
            pl.BlockSpec((1, P2), lambda j, i: (0, 0)),
            pl.BlockSpec((P2, CT), lambda j, i: (0, j)),
            pl.BlockSpec((1, CT), lambda j, i: (0, j)),
        ],
        out_specs=[
            pl.BlockSpec((R, 1), lambda j, i: (i, 0)),
            pl.BlockSpec((R, 1), lambda j, i: (i, 0)),
            pl.BlockSpec((R, P1), lambda j, i: (i, 0)),
            pl.BlockSpec((R, P2), lambda j, i: (i, 0)),
        ],
        out_shape=[
            jax.ShapeDtypeStruct((S, 1), f32),
            jax.ShapeDtypeStruct((S, 1), f32),
            jax.ShapeDtypeStruct((S, P1), jnp.bfloat16),
            jax.ShapeDtypeStruct((S, P2), jnp.bfloat16),
        ],
        scratch_shapes=[
            pltpu.VMEM((S, 1), f32),
            pltpu.VMEM((S, 1), f32),
            pltpu.VMEM((S, 1), f32),
            pltpu.VMEM((S, 1), f32),
            pltpu.VMEM((S, P1), jnp.bfloat16),
            pltpu.VMEM((S, P2), jnp.bfloat16),
        ],
        interpret=interpret,
    )(x, w1ab, b1a2, w1bb, b1b2, w2ab, b2a2, w2bb, b2b2)

    lp1 = HEAD_END - CT * J_T1
    lp2 = T1_END - CT * J_T2
    w1s = jnp.pad(w1bb, ((0, 0), (lp1, 17 * CT - lp1 - V)))
    b1s = jnp.pad(b1b2, ((0, 0), (lp1, 17 * CT - lp1 - V)))
    w2s = jnp.pad(w2bb, ((0, 0), (lp2, 17 * CT - lp2 - V)))
    b2s = jnp.pad(b2b2, ((0, 0), (lp2, 17 * CT - lp2 - V)))

    st = jnp.concatenate([c1, c2, c3, g], axis=1)

    out = pl.pallas_call(
        _out_kernel,
        grid=(JB, I),
        in_specs=[
            pl.BlockSpec((R, D), lambda j, i: (i, 0)),
            pl.BlockSpec((D, CT), lambda j, i: (0, jnp.minimum(j, JH - 1))),
            pl.BlockSpec((1, CT), lambda j, i: (0, jnp.minimum(j, JH - 1))),
            pl.BlockSpec((R, P1), lambda j, i: (i, 0)),
            pl.BlockSpec((P1, CT), lambda j, i: (0, jnp.clip(j - J_T1, 0, 16))),
            pl.BlockSpec((1, CT), lambda j, i: (0, jnp.clip(j - J_T1, 0, 16))),
            pl.BlockSpec((R, P2), lambda j, i: (i, 0)),
            pl.BlockSpec((P2, CT), lambda j, i: (0, jnp.clip(j - J_T2, 0, 16))),
            pl.BlockSpec((1, CT), lambda j, i: (0, jnp.clip(j - J_T2, 0, 16))),
            pl.BlockSpec((R, 5), lambda j, i: (i, 0)),
        ],
        out_specs=pl.BlockSpec((R, CT), lambda j, i: (i, j)),
        out_shape=jax.ShapeDtypeStruct((S, OUT_W), f32),
        interpret=interpret,
    )(x, whb, bh2, p1, w1s, b1s, p2, w2s, b2s, st)

    return out[None, :, :]


def kernel(inp, Wh, bh, W1a, b1a, W1b, b1b, W2a, b2a, W2b, b2b):
    return _run(inp, Wh, bh, W1a, b1a, W1b, b1b, W2a, b2a, W2b, b2b)
